# Initial kernel scaffold; baseline (speedup 1.0000x reference)
import jax
import jax.numpy as jnp
from jax.experimental import pallas as pl


def kernel(x, edge_index, e, fR_W1, fR_b1, fR_W2, fR_b2, fO_W1, fO_b1, fO_W2, fO_b2):
    raise NotImplementedError("write your pallas kernel here")


# speedup vs baseline: 2.8490x; 2.8490x over previous
import functools

import jax
import jax.numpy as jnp
from jax import lax
from jax.experimental import pallas as pl
from jax.experimental.pallas import tpu as pltpu
from jax.experimental.pallas import tpu_sc as plsc

N_NODES = 10000
N_EDGES = 320000
NODE_DIM = 128
EDGE_DIM = 16
HIDDEN = 128

NC = 2
NS = 16
NW = NC * NS

EDGES_PER_W = N_EDGES // NW
GCHUNK = 80
SCHUNK = 80
N_PAD = 10240
ROWS_PER_TILE = N_PAD // NS


def _precompute_body(x_ref, w_ref, b_ref, xd_ref, xs_ref):
    xw = jnp.dot(x_ref[...], w_ref[...], preferred_element_type=jnp.float32)
    xd_ref[...] = xw[:, :NODE_DIM] + b_ref[...]
    xs_ref[...] = xw[:, NODE_DIM:]


def _precompute(x, w_cat, b1):
    blk = 2000
    grid = N_NODES // blk
    return pl.pallas_call(
        _precompute_body,
        grid=(grid,),
        in_specs=[
            pl.BlockSpec((blk, NODE_DIM), lambda i: (i, 0)),
            pl.BlockSpec((NODE_DIM, 2 * NODE_DIM), lambda i: (0, 0)),
            pl.BlockSpec((1, NODE_DIM), lambda i: (0, 0)),
        ],
        out_specs=[
            pl.BlockSpec((blk, NODE_DIM), lambda i: (i, 0)),
            pl.BlockSpec((blk, NODE_DIM), lambda i: (i, 0)),
        ],
        out_shape=[
            jax.ShapeDtypeStruct((N_NODES, NODE_DIM), jnp.float32),
            jax.ShapeDtypeStruct((N_NODES, NODE_DIM), jnp.float32),
        ],
    )(x, w_cat, b1)


def _gather_add_body(xd_hbm, xs_hbm, dst_hbm, src_hbm, g_hbm,
                     idxd_v, idxs_v, bufd, bufs, semd, sems):
    c = lax.axis_index("c")
    s = lax.axis_index("s")
    wid = s * NC + c

    def chunk(i, carry):
        base = pl.multiple_of(wid * EDGES_PER_W + i * GCHUNK, 8)
        pltpu.sync_copy(dst_hbm.at[pl.ds(base, GCHUNK)], idxd_v)
        pltpu.sync_copy(src_hbm.at[pl.ds(base, GCHUNK)], idxs_v)
        cpd = pltpu.async_copy(xd_hbm.at[idxd_v], bufd, semd)
        cps = pltpu.async_copy(xs_hbm.at[idxs_v], bufs, sems)
        cpd.wait()
        cps.wait()

        def addrow(r, carry2):
            for j in range(NODE_DIM // 16):
                sl = pl.ds(j * 16, 16)
                bufd[r, sl] = bufd[r, sl] + bufs[r, sl]
            return carry2

        lax.fori_loop(0, GCHUNK, addrow, 0)
        pltpu.sync_copy(bufd, g_hbm.at[pl.ds(base, GCHUNK)])
        return carry

    lax.fori_loop(0, EDGES_PER_W // GCHUNK, chunk, 0)


def _gather_add(xd, xs, dst, src):
    mesh = plsc.VectorSubcoreMesh(core_axis_name="c", subcore_axis_name="s")
    fn = pl.kernel(
        _gather_add_body,
        mesh=mesh,
        out_type=jax.ShapeDtypeStruct((N_EDGES, NODE_DIM), jnp.float32),
        scratch_types=[
            pltpu.VMEM((GCHUNK,), jnp.int32),
            pltpu.VMEM((GCHUNK,), jnp.int32),
            pltpu.VMEM((GCHUNK, NODE_DIM), jnp.float32),
            pltpu.VMEM((GCHUNK, NODE_DIM), jnp.float32),
            pltpu.SemaphoreType.DMA,
            pltpu.SemaphoreType.DMA,
        ],
    )
    return fn(xd, xs, dst, src)


def _edge_mlp_body(g_ref, e_ref, w1e_ref, w2_ref, b2_ref, out_ref):
    h = g_ref[...] + jnp.dot(e_ref[...], w1e_ref[...],
                             preferred_element_type=jnp.float32)
    h = jnp.maximum(h, 0.0)
    out_ref[...] = jnp.dot(h, w2_ref[...],
                           preferred_element_type=jnp.float32) + b2_ref[...]


def _edge_mlp(g, e, w1e, w2, b2):
    blk = 3200
    grid = N_EDGES // blk
    return pl.pallas_call(
        _edge_mlp_body,
        grid=(grid,),
        in_specs=[
            pl.BlockSpec((blk, HIDDEN), lambda i: (i, 0)),
            pl.BlockSpec((blk, EDGE_DIM), lambda i: (i, 0)),
            pl.BlockSpec((EDGE_DIM, HIDDEN), lambda i: (0, 0)),
            pl.BlockSpec((HIDDEN, EDGE_DIM), lambda i: (0, 0)),
            pl.BlockSpec((1, EDGE_DIM), lambda i: (0, 0)),
        ],
        out_specs=pl.BlockSpec((blk, EDGE_DIM), lambda i: (i, 0)),
        out_shape=jax.ShapeDtypeStruct((N_EDGES, EDGE_DIM), jnp.float32),
    )(g, e, w1e, w2, b2)


def _scatter_add_body(enew_hbm, dst_hbm, out_hbm, idx_v, bufc, bufe, acc_sh):
    c = lax.axis_index("c")
    s = lax.axis_index("s")
    wid = s * NC + c
    row0 = pl.multiple_of(s * ROWS_PER_TILE, 8)

    def zrow(r, carry):
        for j in range(128 // 16):
            bufe[r, pl.ds(j * 16, 16)] = jnp.zeros((16,), jnp.float32)
        return carry

    lax.fori_loop(0, SCHUNK, zrow, 0)
    for k in range(ROWS_PER_TILE // SCHUNK):
        pltpu.sync_copy(bufe, acc_sh.at[pl.ds(row0 + k * SCHUNK, SCHUNK)])
    plsc.subcore_barrier()

    def chunk(i, carry):
        base = pl.multiple_of(wid * EDGES_PER_W + i * SCHUNK, 8)
        pltpu.sync_copy(dst_hbm.at[pl.ds(base, SCHUNK)], idx_v.at[0])
        pltpu.sync_copy(enew_hbm.at[pl.ds(base, SCHUNK)], bufc)

        def crow(r, carry2):
            bufe[r, pl.ds(0, EDGE_DIM)] = bufc[r, :]
            return carry2

        lax.fori_loop(0, SCHUNK, crow, 0)
        pltpu.sync_copy(bufe, acc_sh.at[idx_v.at[0]], add=True)
        return carry

    lax.fori_loop(0, EDGES_PER_W // SCHUNK, chunk, 0)
    plsc.subcore_barrier()
    pltpu.sync_copy(acc_sh.at[pl.ds(row0, ROWS_PER_TILE)],
                    out_hbm.at[c, pl.ds(row0, ROWS_PER_TILE)])


def _scatter_add(e_new, dst):
    mesh = plsc.VectorSubcoreMesh(core_axis_name="c", subcore_axis_name="s")
    fn = pl.kernel(
        _scatter_add_body,
        mesh=mesh,
        out_type=jax.ShapeDtypeStruct((NC, N_PAD, 128), jnp.float32),
        scratch_types=[
            pltpu.VMEM((1, SCHUNK), jnp.int32),
            pltpu.VMEM((SCHUNK, EDGE_DIM), jnp.float32),
            pltpu.VMEM((SCHUNK, 128), jnp.float32),
            pltpu.VMEM_SHARED((N_PAD, 128), jnp.float32),
        ],
    )
    return fn(e_new, dst)


def _node_mlp_body(x_ref, a0_ref, a1_ref, w1x_ref, w1a_ref, b1_ref,
                   w2_ref, b2_ref, out_ref):
    agg = a0_ref[...][:, :EDGE_DIM] + a1_ref[...][:, :EDGE_DIM]
    h = (jnp.dot(x_ref[...], w1x_ref[...], preferred_element_type=jnp.float32)
         + jnp.dot(agg, w1a_ref[...], preferred_element_type=jnp.float32)
         + b1_ref[...])
    h = jnp.maximum(h, 0.0)
    out_ref[...] = jnp.dot(h, w2_ref[...],
                           preferred_element_type=jnp.float32) + b2_ref[...]


def _node_mlp(x, a0, a1, w1x, w1a, b1, w2, b2):
    blk = 2000
    grid = N_NODES // blk
    return pl.pallas_call(
        _node_mlp_body,
        grid=(grid,),
        in_specs=[
            pl.BlockSpec((blk, NODE_DIM), lambda i: (i, 0)),
            pl.BlockSpec((blk, 128), lambda i: (i, 0)),
            pl.BlockSpec((blk, 128), lambda i: (i, 0)),
            pl.BlockSpec((NODE_DIM, HIDDEN), lambda i: (0, 0)),
            pl.BlockSpec((EDGE_DIM, HIDDEN), lambda i: (0, 0)),
            pl.BlockSpec((1, HIDDEN), lambda i: (0, 0)),
            pl.BlockSpec((HIDDEN, NODE_DIM), lambda i: (0, 0)),
            pl.BlockSpec((1, NODE_DIM), lambda i: (0, 0)),
        ],
        out_specs=pl.BlockSpec((blk, NODE_DIM), lambda i: (i, 0)),
        out_shape=jax.ShapeDtypeStruct((N_NODES, NODE_DIM), jnp.float32),
    )(x, a0, a1, w1x, w1a, b1, w2, b2)


def kernel(x, edge_index, e, fR_W1, fR_b1, fR_W2, fR_b2,
           fO_W1, fO_b1, fO_W2, fO_b2):
    src = edge_index[0].astype(jnp.int32)
    dst = edge_index[1].astype(jnp.int32)

    w_cat = jnp.concatenate(
        [fR_W1[:NODE_DIM], fR_W1[NODE_DIM:2 * NODE_DIM]], axis=1)
    xd, xs = _precompute(x, w_cat, fR_b1.reshape(1, -1))

    g = _gather_add(xd, xs, dst, src)

    e_new = _edge_mlp(g, e, fR_W1[2 * NODE_DIM:], fR_W2, fR_b2.reshape(1, -1))

    partials = _scatter_add(e_new, dst)

    x_new = _node_mlp(x, partials[0], partials[1],
                      fO_W1[:NODE_DIM], fO_W1[NODE_DIM:],
                      fO_b1.reshape(1, -1), fO_W2, fO_b2.reshape(1, -1))
    return (x_new, e_new)


# speedup vs baseline: 3.4189x; 1.2000x over previous
import functools

import jax
import jax.numpy as jnp
from jax import lax
from jax.experimental import pallas as pl
from jax.experimental.pallas import tpu as pltpu
from jax.experimental.pallas import tpu_sc as plsc

N_NODES = 10000
N_EDGES = 320000
NODE_DIM = 128
EDGE_DIM = 16
HIDDEN = 128

NC = 2
NS = 16
NW = NC * NS

EDGES_PER_W = N_EDGES // NW
GCHUNK = 80
SCHUNK = 80
N_PAD = 10240
ROWS_PER_TILE = N_PAD // NS


def _precompute_body(x_ref, w_ref, b_ref, xd_ref, xs_ref):
    xw = jnp.dot(x_ref[...], w_ref[...], preferred_element_type=jnp.float32)
    xd_ref[...] = xw[:, :NODE_DIM] + b_ref[...]
    xs_ref[...] = xw[:, NODE_DIM:]


def _precompute(x, w_cat, b1):
    blk = 2000
    grid = N_NODES // blk
    return pl.pallas_call(
        _precompute_body,
        grid=(grid,),
        in_specs=[
            pl.BlockSpec((blk, NODE_DIM), lambda i: (i, 0)),
            pl.BlockSpec((NODE_DIM, 2 * NODE_DIM), lambda i: (0, 0)),
            pl.BlockSpec((1, NODE_DIM), lambda i: (0, 0)),
        ],
        out_specs=[
            pl.BlockSpec((blk, NODE_DIM), lambda i: (i, 0)),
            pl.BlockSpec((blk, NODE_DIM), lambda i: (i, 0)),
        ],
        out_shape=[
            jax.ShapeDtypeStruct((N_NODES, NODE_DIM), jnp.float32),
            jax.ShapeDtypeStruct((N_NODES, NODE_DIM), jnp.float32),
        ],
    )(x, w_cat, b1)


NCHUNK = EDGES_PER_W // GCHUNK


def _gather_add_body(xd_hbm, xs_hbm, dst_hbm, src_hbm, g_hbm,
                     idxd_all, idxs_all,
                     bufd0, bufs0, bufd1, bufs1,
                     semd0, sems0, semd1, sems1, wsem0, wsem1):
    c = lax.axis_index("c")
    s = lax.axis_index("s")
    wid = s * NC + c

    pltpu.sync_copy(dst_hbm.at[wid], idxd_all)
    pltpu.sync_copy(src_hbm.at[wid], idxs_all)

    def add_rows(bd, bs):
        def addrow(r, carry2):
            for j in range(NODE_DIM // 16):
                sl = pl.ds(j * 16, 16)
                bd[r, sl] = bd[r, sl] + bs[r, sl]
            return carry2
        lax.fori_loop(0, GCHUNK, addrow, 0)

    def start_gather(ci, bd, bs, sd, ss):
        pltpu.async_copy(xd_hbm.at[idxd_all.at[ci]], bd, sd)
        pltpu.async_copy(xs_hbm.at[idxs_all.at[ci]], bs, ss)

    def gbase(ci):
        return pl.multiple_of(wid * EDGES_PER_W + ci * GCHUNK, 8)

    def pair(i, carry):
        ca = 2 * i
        cb = 2 * i + 1
        cpd0 = pltpu.async_copy(xd_hbm.at[idxd_all.at[ca]], bufd0, semd0)
        cps0 = pltpu.async_copy(xs_hbm.at[idxs_all.at[ca]], bufs0, sems0)
        cpd1 = pltpu.async_copy(xd_hbm.at[idxd_all.at[cb]], bufd1, semd1)
        cps1 = pltpu.async_copy(xs_hbm.at[idxs_all.at[cb]], bufs1, sems1)
        cpd0.wait()
        cps0.wait()
        add_rows(bufd0, bufs0)
        w0 = pltpu.async_copy(bufd0, g_hbm.at[pl.ds(gbase(ca), GCHUNK)], wsem0)
        cpd1.wait()
        cps1.wait()
        add_rows(bufd1, bufs1)
        w1 = pltpu.async_copy(bufd1, g_hbm.at[pl.ds(gbase(cb), GCHUNK)], wsem1)
        w0.wait()
        w1.wait()
        return carry

    lax.fori_loop(0, NCHUNK // 2, pair, 0)

    clast = NCHUNK - 1
    cpd0 = pltpu.async_copy(xd_hbm.at[idxd_all.at[clast]], bufd0, semd0)
    cps0 = pltpu.async_copy(xs_hbm.at[idxs_all.at[clast]], bufs0, sems0)
    cpd0.wait()
    cps0.wait()
    add_rows(bufd0, bufs0)
    pltpu.sync_copy(bufd0, g_hbm.at[pl.ds(gbase(clast), GCHUNK)])


def _gather_add(xd, xs, dst, src):
    dst_r = dst.reshape(NW, NCHUNK, GCHUNK)
    src_r = src.reshape(NW, NCHUNK, GCHUNK)
    mesh = plsc.VectorSubcoreMesh(core_axis_name="c", subcore_axis_name="s")
    fn = pl.kernel(
        _gather_add_body,
        mesh=mesh,
        out_type=jax.ShapeDtypeStruct((N_EDGES, NODE_DIM), jnp.float32),
        scratch_types=[
            pltpu.VMEM((NCHUNK, GCHUNK), jnp.int32),
            pltpu.VMEM((NCHUNK, GCHUNK), jnp.int32),
            pltpu.VMEM((GCHUNK, NODE_DIM), jnp.float32),
            pltpu.VMEM((GCHUNK, NODE_DIM), jnp.float32),
            pltpu.VMEM((GCHUNK, NODE_DIM), jnp.float32),
            pltpu.VMEM((GCHUNK, NODE_DIM), jnp.float32),
            pltpu.SemaphoreType.DMA,
            pltpu.SemaphoreType.DMA,
            pltpu.SemaphoreType.DMA,
            pltpu.SemaphoreType.DMA,
            pltpu.SemaphoreType.DMA,
            pltpu.SemaphoreType.DMA,
        ],
    )
    return fn(xd, xs, dst_r, src_r)


def _edge_mlp_body(g_ref, e_ref, w1e_ref, w2_ref, b2_ref, out_ref):
    h = g_ref[...] + jnp.dot(e_ref[...], w1e_ref[...],
                             preferred_element_type=jnp.float32)
    h = jnp.maximum(h, 0.0)
    out_ref[...] = jnp.dot(h, w2_ref[...],
                           preferred_element_type=jnp.float32) + b2_ref[...]


def _edge_mlp(g, e, w1e, w2, b2):
    blk = 3200
    grid = N_EDGES // blk
    return pl.pallas_call(
        _edge_mlp_body,
        grid=(grid,),
        in_specs=[
            pl.BlockSpec((blk, HIDDEN), lambda i: (i, 0)),
            pl.BlockSpec((blk, EDGE_DIM), lambda i: (i, 0)),
            pl.BlockSpec((EDGE_DIM, HIDDEN), lambda i: (0, 0)),
            pl.BlockSpec((HIDDEN, EDGE_DIM), lambda i: (0, 0)),
            pl.BlockSpec((1, EDGE_DIM), lambda i: (0, 0)),
        ],
        out_specs=pl.BlockSpec((blk, EDGE_DIM), lambda i: (i, 0)),
        out_shape=jax.ShapeDtypeStruct((N_EDGES, EDGE_DIM), jnp.float32),
    )(g, e, w1e, w2, b2)


def _scatter_add_body(enew_hbm, dst_hbm, out_hbm, idx_v, bufc, bufe, acc_sh):
    c = lax.axis_index("c")
    s = lax.axis_index("s")
    wid = s * NC + c
    row0 = pl.multiple_of(s * ROWS_PER_TILE, 8)

    def zrow(r, carry):
        for j in range(128 // 16):
            bufe[r, pl.ds(j * 16, 16)] = jnp.zeros((16,), jnp.float32)
        return carry

    lax.fori_loop(0, SCHUNK, zrow, 0)
    for k in range(ROWS_PER_TILE // SCHUNK):
        pltpu.sync_copy(bufe, acc_sh.at[pl.ds(row0 + k * SCHUNK, SCHUNK)])
    plsc.subcore_barrier()

    def chunk(i, carry):
        base = pl.multiple_of(wid * EDGES_PER_W + i * SCHUNK, 8)
        pltpu.sync_copy(dst_hbm.at[pl.ds(base, SCHUNK)], idx_v.at[0])
        pltpu.sync_copy(enew_hbm.at[pl.ds(base, SCHUNK)], bufc)

        def crow(r, carry2):
            bufe[r, pl.ds(0, EDGE_DIM)] = bufc[r, :]
            return carry2

        lax.fori_loop(0, SCHUNK, crow, 0)
        pltpu.sync_copy(bufe, acc_sh.at[idx_v.at[0]], add=True)
        return carry

    lax.fori_loop(0, EDGES_PER_W // SCHUNK, chunk, 0)
    plsc.subcore_barrier()
    pltpu.sync_copy(acc_sh.at[pl.ds(row0, ROWS_PER_TILE)],
                    out_hbm.at[c, pl.ds(row0, ROWS_PER_TILE)])


def _scatter_add(e_new, dst):
    mesh = plsc.VectorSubcoreMesh(core_axis_name="c", subcore_axis_name="s")
    fn = pl.kernel(
        _scatter_add_body,
        mesh=mesh,
        out_type=jax.ShapeDtypeStruct((NC, N_PAD, 128), jnp.float32),
        scratch_types=[
            pltpu.VMEM((1, SCHUNK), jnp.int32),
            pltpu.VMEM((SCHUNK, EDGE_DIM), jnp.float32),
            pltpu.VMEM((SCHUNK, 128), jnp.float32),
            pltpu.VMEM_SHARED((N_PAD, 128), jnp.float32),
        ],
    )
    return fn(e_new, dst)


def _node_mlp_body(x_ref, a0_ref, a1_ref, w1x_ref, w1a_ref, b1_ref,
                   w2_ref, b2_ref, out_ref):
    agg = a0_ref[...][:, :EDGE_DIM] + a1_ref[...][:, :EDGE_DIM]
    h = (jnp.dot(x_ref[...], w1x_ref[...], preferred_element_type=jnp.float32)
         + jnp.dot(agg, w1a_ref[...], preferred_element_type=jnp.float32)
         + b1_ref[...])
    h = jnp.maximum(h, 0.0)
    out_ref[...] = jnp.dot(h, w2_ref[...],
                           preferred_element_type=jnp.float32) + b2_ref[...]


def _node_mlp(x, a0, a1, w1x, w1a, b1, w2, b2):
    blk = 2000
    grid = N_NODES // blk
    return pl.pallas_call(
        _node_mlp_body,
        grid=(grid,),
        in_specs=[
            pl.BlockSpec((blk, NODE_DIM), lambda i: (i, 0)),
            pl.BlockSpec((blk, 128), lambda i: (i, 0)),
            pl.BlockSpec((blk, 128), lambda i: (i, 0)),
            pl.BlockSpec((NODE_DIM, HIDDEN), lambda i: (0, 0)),
            pl.BlockSpec((EDGE_DIM, HIDDEN), lambda i: (0, 0)),
            pl.BlockSpec((1, HIDDEN), lambda i: (0, 0)),
            pl.BlockSpec((HIDDEN, NODE_DIM), lambda i: (0, 0)),
            pl.BlockSpec((1, NODE_DIM), lambda i: (0, 0)),
        ],
        out_specs=pl.BlockSpec((blk, NODE_DIM), lambda i: (i, 0)),
        out_shape=jax.ShapeDtypeStruct((N_NODES, NODE_DIM), jnp.float32),
    )(x, a0, a1, w1x, w1a, b1, w2, b2)


def kernel(x, edge_index, e, fR_W1, fR_b1, fR_W2, fR_b2,
           fO_W1, fO_b1, fO_W2, fO_b2):
    src = edge_index[0].astype(jnp.int32)
    dst = edge_index[1].astype(jnp.int32)

    w_cat = jnp.concatenate(
        [fR_W1[:NODE_DIM], fR_W1[NODE_DIM:2 * NODE_DIM]], axis=1)
    xd, xs = _precompute(x, w_cat, fR_b1.reshape(1, -1))

    g = _gather_add(xd, xs, dst, src)

    e_new = _edge_mlp(g, e, fR_W1[2 * NODE_DIM:], fR_W2, fR_b2.reshape(1, -1))

    partials = _scatter_add(e_new, dst)

    x_new = _node_mlp(x, partials[0], partials[1],
                      fO_W1[:NODE_DIM], fO_W1[NODE_DIM:],
                      fO_b1.reshape(1, -1), fO_W2, fO_b2.reshape(1, -1))
    return (x_new, e_new)


# speedup vs baseline: 3.9229x; 1.1474x over previous
import functools

import jax
import jax.numpy as jnp
from jax import lax
from jax.experimental import pallas as pl
from jax.experimental.pallas import tpu as pltpu
from jax.experimental.pallas import tpu_sc as plsc

N_NODES = 10000
N_EDGES = 320000
NODE_DIM = 128
EDGE_DIM = 16
HIDDEN = 128

NC = 2
NS = 16
NW = NC * NS

EDGES_PER_W = N_EDGES // NW
GCHUNK = 80
SCHUNK = 80
N_PAD = 10240
ROWS_PER_TILE = N_PAD // NS


def _precompute_body(x_ref, w_ref, b_ref, xd_ref, xs_ref):
    xw = jnp.dot(x_ref[...], w_ref[...], preferred_element_type=jnp.float32)
    xd_ref[...] = xw[:, :NODE_DIM] + b_ref[...]
    xs_ref[...] = xw[:, NODE_DIM:]


def _precompute(x, w_cat, b1):
    blk = 2000
    grid = N_NODES // blk
    return pl.pallas_call(
        _precompute_body,
        grid=(grid,),
        in_specs=[
            pl.BlockSpec((blk, NODE_DIM), lambda i: (i, 0)),
            pl.BlockSpec((NODE_DIM, 2 * NODE_DIM), lambda i: (0, 0)),
            pl.BlockSpec((1, NODE_DIM), lambda i: (0, 0)),
        ],
        out_specs=[
            pl.BlockSpec((blk, NODE_DIM), lambda i: (i, 0)),
            pl.BlockSpec((blk, NODE_DIM), lambda i: (i, 0)),
        ],
        out_shape=[
            jax.ShapeDtypeStruct((N_NODES, NODE_DIM), jnp.float32),
            jax.ShapeDtypeStruct((N_NODES, NODE_DIM), jnp.float32),
        ],
    )(x, w_cat, b1)


NCHUNK = EDGES_PER_W // GCHUNK


def _gather_add_body(xd_hbm, xs_hbm, dst_hbm, src_hbm, g_hbm,
                     idxd_all, idxs_all,
                     bufd0, bufs0, bufd1, bufs1,
                     semd0, sems0, semd1, sems1, wsem0, wsem1):
    c = lax.axis_index("c")
    s = lax.axis_index("s")
    wid = s * NC + c

    pltpu.sync_copy(dst_hbm.at[wid], idxd_all)
    pltpu.sync_copy(src_hbm.at[wid], idxs_all)

    def add_rows(bd, bs):
        def addrow(r, carry2):
            for j in range(NODE_DIM // 16):
                sl = pl.ds(j * 16, 16)
                bd[r, sl] = bd[r, sl] + bs[r, sl]
            return carry2
        lax.fori_loop(0, GCHUNK, addrow, 0)

    def start_gather(ci, bd, bs, sd, ss):
        pltpu.async_copy(xd_hbm.at[idxd_all.at[ci]], bd, sd)
        pltpu.async_copy(xs_hbm.at[idxs_all.at[ci]], bs, ss)

    def gbase(ci):
        return pl.multiple_of(wid * EDGES_PER_W + ci * GCHUNK, 8)

    def pair(i, carry):
        ca = 2 * i
        cb = 2 * i + 1
        cpd0 = pltpu.async_copy(xd_hbm.at[idxd_all.at[ca]], bufd0, semd0)
        cps0 = pltpu.async_copy(xs_hbm.at[idxs_all.at[ca]], bufs0, sems0)
        cpd1 = pltpu.async_copy(xd_hbm.at[idxd_all.at[cb]], bufd1, semd1)
        cps1 = pltpu.async_copy(xs_hbm.at[idxs_all.at[cb]], bufs1, sems1)
        cpd0.wait()
        cps0.wait()
        add_rows(bufd0, bufs0)
        w0 = pltpu.async_copy(bufd0, g_hbm.at[pl.ds(gbase(ca), GCHUNK)], wsem0)
        cpd1.wait()
        cps1.wait()
        add_rows(bufd1, bufs1)
        w1 = pltpu.async_copy(bufd1, g_hbm.at[pl.ds(gbase(cb), GCHUNK)], wsem1)
        w0.wait()
        w1.wait()
        return carry

    lax.fori_loop(0, NCHUNK // 2, pair, 0)

    clast = NCHUNK - 1
    cpd0 = pltpu.async_copy(xd_hbm.at[idxd_all.at[clast]], bufd0, semd0)
    cps0 = pltpu.async_copy(xs_hbm.at[idxs_all.at[clast]], bufs0, sems0)
    cpd0.wait()
    cps0.wait()
    add_rows(bufd0, bufs0)
    pltpu.sync_copy(bufd0, g_hbm.at[pl.ds(gbase(clast), GCHUNK)])


def _gather_add(xd, xs, dst, src):
    dst_r = dst.reshape(NW, NCHUNK, GCHUNK)
    src_r = src.reshape(NW, NCHUNK, GCHUNK)
    mesh = plsc.VectorSubcoreMesh(core_axis_name="c", subcore_axis_name="s")
    fn = pl.kernel(
        _gather_add_body,
        mesh=mesh,
        out_type=jax.ShapeDtypeStruct((N_EDGES, NODE_DIM), jnp.float32),
        scratch_types=[
            pltpu.VMEM((NCHUNK, GCHUNK), jnp.int32),
            pltpu.VMEM((NCHUNK, GCHUNK), jnp.int32),
            pltpu.VMEM((GCHUNK, NODE_DIM), jnp.float32),
            pltpu.VMEM((GCHUNK, NODE_DIM), jnp.float32),
            pltpu.VMEM((GCHUNK, NODE_DIM), jnp.float32),
            pltpu.VMEM((GCHUNK, NODE_DIM), jnp.float32),
            pltpu.SemaphoreType.DMA,
            pltpu.SemaphoreType.DMA,
            pltpu.SemaphoreType.DMA,
            pltpu.SemaphoreType.DMA,
            pltpu.SemaphoreType.DMA,
            pltpu.SemaphoreType.DMA,
        ],
    )
    return fn(xd, xs, dst_r, src_r)


def _edge_mlp_body(g_ref, e_ref, w1e_ref, w2_ref, b2_ref, out_ref):
    h = g_ref[...] + jnp.dot(e_ref[...], w1e_ref[...],
                             preferred_element_type=jnp.float32)
    h = jnp.maximum(h, 0.0)
    out_ref[...] = jnp.dot(h, w2_ref[...],
                           preferred_element_type=jnp.float32) + b2_ref[...]


def _edge_mlp(g, e, w1e, w2, b2):
    blk = 3200
    grid = N_EDGES // blk
    return pl.pallas_call(
        _edge_mlp_body,
        grid=(grid,),
        in_specs=[
            pl.BlockSpec((blk, HIDDEN), lambda i: (i, 0)),
            pl.BlockSpec((blk, EDGE_DIM), lambda i: (i, 0)),
            pl.BlockSpec((EDGE_DIM, HIDDEN), lambda i: (0, 0)),
            pl.BlockSpec((HIDDEN, EDGE_DIM), lambda i: (0, 0)),
            pl.BlockSpec((1, EDGE_DIM), lambda i: (0, 0)),
        ],
        out_specs=pl.BlockSpec((blk, EDGE_DIM), lambda i: (i, 0)),
        out_shape=jax.ShapeDtypeStruct((N_EDGES, EDGE_DIM), jnp.float32),
    )(g, e, w1e, w2, b2)


def _scatter_add_body(enew_hbm, dst_hbm, out_hbm, idx_all,
                      bufc0, bufc1, bufe0,
                      csem0, csem1, acc_sh):
    c = lax.axis_index("c")
    s = lax.axis_index("s")
    wid = s * NC + c
    row0 = pl.multiple_of(s * ROWS_PER_TILE, 8)

    def zrow(r, carry):
        for j in range(128 // 16):
            bufe0[r, pl.ds(j * 16, 16)] = jnp.zeros((16,), jnp.float32)
        return carry

    lax.fori_loop(0, SCHUNK, zrow, 0)
    for k in range(ROWS_PER_TILE // SCHUNK):
        pltpu.sync_copy(bufe0, acc_sh.at[pl.ds(row0 + k * SCHUNK, SCHUNK)])
    pltpu.sync_copy(dst_hbm.at[wid], idx_all)
    plsc.subcore_barrier()

    def expand(bc, be):
        def crow(r, carry2):
            be[r, pl.ds(0, EDGE_DIM)] = bc[r, :]
            return carry2
        lax.fori_loop(0, SCHUNK, crow, 0)

    def pair(i, carry):
        ca = 2 * i
        cb = 2 * i + 1
        la = pltpu.async_copy(enew_hbm.at[wid, ca], bufc0, csem0)
        lb = pltpu.async_copy(enew_hbm.at[wid, cb], bufc1, csem1)
        la.wait()
        expand(bufc0, bufe0)
        pltpu.sync_copy(bufe0, acc_sh.at[idx_all.at[ca]], add=True)
        lb.wait()
        expand(bufc1, bufe0)
        pltpu.sync_copy(bufe0, acc_sh.at[idx_all.at[cb]], add=True)
        return carry

    lax.fori_loop(0, NCHUNK // 2, pair, 0)

    clast = NCHUNK - 1
    la = pltpu.async_copy(enew_hbm.at[wid, clast], bufc0, csem0)
    la.wait()
    expand(bufc0, bufe0)
    pltpu.sync_copy(bufe0, acc_sh.at[idx_all.at[clast]], add=True)

    plsc.subcore_barrier()
    pltpu.sync_copy(acc_sh.at[pl.ds(row0, ROWS_PER_TILE)],
                    out_hbm.at[c, pl.ds(row0, ROWS_PER_TILE)])


def _scatter_add(e_new, dst):
    enew_r = e_new.reshape(NW, NCHUNK, SCHUNK, EDGE_DIM)
    dst_r = dst.reshape(NW, NCHUNK, SCHUNK)
    mesh = plsc.VectorSubcoreMesh(core_axis_name="c", subcore_axis_name="s")
    fn = pl.kernel(
        _scatter_add_body,
        mesh=mesh,
        out_type=jax.ShapeDtypeStruct((NC, N_PAD, 128), jnp.float32),
        scratch_types=[
            pltpu.VMEM((NCHUNK, SCHUNK), jnp.int32),
            pltpu.VMEM((SCHUNK, EDGE_DIM), jnp.float32),
            pltpu.VMEM((SCHUNK, EDGE_DIM), jnp.float32),
            pltpu.VMEM((SCHUNK, 128), jnp.float32),
            pltpu.SemaphoreType.DMA,
            pltpu.SemaphoreType.DMA,
            pltpu.VMEM_SHARED((N_PAD, 128), jnp.float32),
        ],
    )
    return fn(enew_r, dst_r)


def _node_mlp_body(x_ref, a0_ref, a1_ref, w1x_ref, w1a_ref, b1_ref,
                   w2_ref, b2_ref, out_ref):
    agg = a0_ref[...][:, :EDGE_DIM] + a1_ref[...][:, :EDGE_DIM]
    h = (jnp.dot(x_ref[...], w1x_ref[...], preferred_element_type=jnp.float32)
         + jnp.dot(agg, w1a_ref[...], preferred_element_type=jnp.float32)
         + b1_ref[...])
    h = jnp.maximum(h, 0.0)
    out_ref[...] = jnp.dot(h, w2_ref[...],
                           preferred_element_type=jnp.float32) + b2_ref[...]


def _node_mlp(x, a0, a1, w1x, w1a, b1, w2, b2):
    blk = 2000
    grid = N_NODES // blk
    return pl.pallas_call(
        _node_mlp_body,
        grid=(grid,),
        in_specs=[
            pl.BlockSpec((blk, NODE_DIM), lambda i: (i, 0)),
            pl.BlockSpec((blk, 128), lambda i: (i, 0)),
            pl.BlockSpec((blk, 128), lambda i: (i, 0)),
            pl.BlockSpec((NODE_DIM, HIDDEN), lambda i: (0, 0)),
            pl.BlockSpec((EDGE_DIM, HIDDEN), lambda i: (0, 0)),
            pl.BlockSpec((1, HIDDEN), lambda i: (0, 0)),
            pl.BlockSpec((HIDDEN, NODE_DIM), lambda i: (0, 0)),
            pl.BlockSpec((1, NODE_DIM), lambda i: (0, 0)),
        ],
        out_specs=pl.BlockSpec((blk, NODE_DIM), lambda i: (i, 0)),
        out_shape=jax.ShapeDtypeStruct((N_NODES, NODE_DIM), jnp.float32),
    )(x, a0, a1, w1x, w1a, b1, w2, b2)


def kernel(x, edge_index, e, fR_W1, fR_b1, fR_W2, fR_b2,
           fO_W1, fO_b1, fO_W2, fO_b2):
    src = edge_index[0].astype(jnp.int32)
    dst = edge_index[1].astype(jnp.int32)

    w_cat = jnp.concatenate(
        [fR_W1[:NODE_DIM], fR_W1[NODE_DIM:2 * NODE_DIM]], axis=1)
    xd, xs = _precompute(x, w_cat, fR_b1.reshape(1, -1))

    g = _gather_add(xd, xs, dst, src)

    e_new = _edge_mlp(g, e, fR_W1[2 * NODE_DIM:], fR_W2, fR_b2.reshape(1, -1))

    partials = _scatter_add(e_new, dst)

    x_new = _node_mlp(x, partials[0], partials[1],
                      fO_W1[:NODE_DIM], fO_W1[NODE_DIM:],
                      fO_b1.reshape(1, -1), fO_W2, fO_b2.reshape(1, -1))
    return (x_new, e_new)


# speedup vs baseline: 4.1089x; 1.0474x over previous
import functools

import jax
import jax.numpy as jnp
from jax import lax
from jax.experimental import pallas as pl
from jax.experimental.pallas import tpu as pltpu
from jax.experimental.pallas import tpu_sc as plsc

N_NODES = 10000
N_EDGES = 320000
NODE_DIM = 128
EDGE_DIM = 16
HIDDEN = 128

NC = 2
NS = 16
NW = NC * NS

EDGES_PER_W = N_EDGES // NW
GCHUNK = 80
SCHUNK = 80
N_PAD = 10240
ROWS_PER_TILE = N_PAD // NS


def _precompute_body(x_ref, w_ref, b_ref, xd_ref, xs_ref):
    xw = jnp.dot(x_ref[...], w_ref[...], preferred_element_type=jnp.float32)
    xd_ref[...] = xw[:, :NODE_DIM] + b_ref[...]
    xs_ref[...] = xw[:, NODE_DIM:]


def _precompute(x, w_cat, b1):
    blk = 2000
    grid = N_NODES // blk
    return pl.pallas_call(
        _precompute_body,
        grid=(grid,),
        in_specs=[
            pl.BlockSpec((blk, NODE_DIM), lambda i: (i, 0)),
            pl.BlockSpec((NODE_DIM, 2 * NODE_DIM), lambda i: (0, 0)),
            pl.BlockSpec((1, NODE_DIM), lambda i: (0, 0)),
        ],
        out_specs=[
            pl.BlockSpec((blk, NODE_DIM), lambda i: (i, 0)),
            pl.BlockSpec((blk, NODE_DIM), lambda i: (i, 0)),
        ],
        out_shape=[
            jax.ShapeDtypeStruct((N_NODES, NODE_DIM), jnp.float32),
            jax.ShapeDtypeStruct((N_NODES, NODE_DIM), jnp.float32),
        ],
    )(x, w_cat, b1)


NCHUNK = EDGES_PER_W // GCHUNK


def _gather_add_body(xd_hbm, xs_hbm, dst_hbm, src_hbm, g_hbm,
                     idxd_all, idxs_all,
                     bufd0, bufs0, bufd1, bufs1, bufd2, bufs2, bufd3, bufs3,
                     semd0, sems0, semd1, sems1, semd2, sems2, semd3, sems3,
                     wsem0, wsem1, wsem2, wsem3):
    c = lax.axis_index("c")
    s = lax.axis_index("s")
    wid = s * NC + c

    pltpu.sync_copy(dst_hbm.at[wid], idxd_all)
    pltpu.sync_copy(src_hbm.at[wid], idxs_all)

    def add_rows(bd, bs):
        def addrow(r, carry2):
            for j in range(NODE_DIM // 16):
                sl = pl.ds(j * 16, 16)
                bd[r, sl] = bd[r, sl] + bs[r, sl]
            return carry2
        lax.fori_loop(0, GCHUNK, addrow, 0)

    def gbase(ci):
        return pl.multiple_of(wid * EDGES_PER_W + ci * GCHUNK, 8)

    sets = ((bufd0, bufs0, semd0, sems0, wsem0),
            (bufd1, bufs1, semd1, sems1, wsem1),
            (bufd2, bufs2, semd2, sems2, wsem2),
            (bufd3, bufs3, semd3, sems3, wsem3))

    def quad(i, carry):
        gathers = []
        for k, (bd, bs, sd, ss, _) in enumerate(sets):
            ci = 4 * i + k
            gathers.append((pltpu.async_copy(xd_hbm.at[idxd_all.at[ci]], bd, sd),
                            pltpu.async_copy(xs_hbm.at[idxs_all.at[ci]], bs, ss)))
        writes = []
        for k, (bd, bs, _, _, ws) in enumerate(sets):
            ci = 4 * i + k
            gathers[k][0].wait()
            gathers[k][1].wait()
            add_rows(bd, bs)
            writes.append(pltpu.async_copy(
                bd, g_hbm.at[pl.ds(gbase(ci), GCHUNK)], ws))
        for w in writes:
            w.wait()
        return carry

    lax.fori_loop(0, NCHUNK // 4, quad, 0)

    clast = NCHUNK - 1
    cpd0 = pltpu.async_copy(xd_hbm.at[idxd_all.at[clast]], bufd0, semd0)
    cps0 = pltpu.async_copy(xs_hbm.at[idxs_all.at[clast]], bufs0, sems0)
    cpd0.wait()
    cps0.wait()
    add_rows(bufd0, bufs0)
    pltpu.sync_copy(bufd0, g_hbm.at[pl.ds(gbase(clast), GCHUNK)])


def _gather_add(xd, xs, dst, src):
    dst_r = dst.reshape(NW, NCHUNK, GCHUNK)
    src_r = src.reshape(NW, NCHUNK, GCHUNK)
    mesh = plsc.VectorSubcoreMesh(core_axis_name="c", subcore_axis_name="s")
    fn = pl.kernel(
        _gather_add_body,
        mesh=mesh,
        out_type=jax.ShapeDtypeStruct((N_EDGES, NODE_DIM), jnp.float32),
        scratch_types=[
            pltpu.VMEM((NCHUNK, GCHUNK), jnp.int32),
            pltpu.VMEM((NCHUNK, GCHUNK), jnp.int32),
        ] + [pltpu.VMEM((GCHUNK, NODE_DIM), jnp.float32)] * 8
          + [pltpu.SemaphoreType.DMA] * 12,
    )
    return fn(xd, xs, dst_r, src_r)


def _edge_mlp_body(g_ref, e_ref, w1e_ref, w2_ref, b2_ref, out_ref):
    h = g_ref[...] + jnp.dot(e_ref[...], w1e_ref[...],
                             preferred_element_type=jnp.float32)
    h = jnp.maximum(h, 0.0)
    out_ref[...] = jnp.dot(h, w2_ref[...],
                           preferred_element_type=jnp.float32) + b2_ref[...]


def _edge_mlp(g, e, w1e, w2, b2):
    blk = 3200
    grid = N_EDGES // blk
    return pl.pallas_call(
        _edge_mlp_body,
        grid=(grid,),
        in_specs=[
            pl.BlockSpec((blk, HIDDEN), lambda i: (i, 0)),
            pl.BlockSpec((blk, EDGE_DIM), lambda i: (i, 0)),
            pl.BlockSpec((EDGE_DIM, HIDDEN), lambda i: (0, 0)),
            pl.BlockSpec((HIDDEN, EDGE_DIM), lambda i: (0, 0)),
            pl.BlockSpec((1, EDGE_DIM), lambda i: (0, 0)),
        ],
        out_specs=pl.BlockSpec((blk, EDGE_DIM), lambda i: (i, 0)),
        out_shape=jax.ShapeDtypeStruct((N_EDGES, EDGE_DIM), jnp.float32),
    )(g, e, w1e, w2, b2)


def _scatter_add_body(enew_hbm, dst_hbm, out_hbm, idx_all,
                      bufc0, bufc1, bufe0,
                      csem0, csem1, acc_sh):
    c = lax.axis_index("c")
    s = lax.axis_index("s")
    wid = s * NC + c
    row0 = pl.multiple_of(s * ROWS_PER_TILE, 8)

    def zrow(r, carry):
        for j in range(128 // 16):
            bufe0[r, pl.ds(j * 16, 16)] = jnp.zeros((16,), jnp.float32)
        return carry

    lax.fori_loop(0, SCHUNK, zrow, 0)
    for k in range(ROWS_PER_TILE // SCHUNK):
        pltpu.sync_copy(bufe0, acc_sh.at[pl.ds(row0 + k * SCHUNK, SCHUNK)])
    pltpu.sync_copy(dst_hbm.at[wid], idx_all)
    plsc.subcore_barrier()

    def expand(bc, be):
        def crow(r, carry2):
            be[r, pl.ds(0, EDGE_DIM)] = bc[r, :]
            return carry2
        lax.fori_loop(0, SCHUNK, crow, 0)

    def pair(i, carry):
        ca = 2 * i
        cb = 2 * i + 1
        la = pltpu.async_copy(enew_hbm.at[wid, ca], bufc0, csem0)
        lb = pltpu.async_copy(enew_hbm.at[wid, cb], bufc1, csem1)
        la.wait()
        expand(bufc0, bufe0)
        pltpu.sync_copy(bufe0, acc_sh.at[idx_all.at[ca]], add=True)
        lb.wait()
        expand(bufc1, bufe0)
        pltpu.sync_copy(bufe0, acc_sh.at[idx_all.at[cb]], add=True)
        return carry

    lax.fori_loop(0, NCHUNK // 2, pair, 0)

    clast = NCHUNK - 1
    la = pltpu.async_copy(enew_hbm.at[wid, clast], bufc0, csem0)
    la.wait()
    expand(bufc0, bufe0)
    pltpu.sync_copy(bufe0, acc_sh.at[idx_all.at[clast]], add=True)

    plsc.subcore_barrier()
    pltpu.sync_copy(acc_sh.at[pl.ds(row0, ROWS_PER_TILE)],
                    out_hbm.at[c, pl.ds(row0, ROWS_PER_TILE)])


def _scatter_add(e_new, dst):
    enew_r = e_new.reshape(NW, NCHUNK, SCHUNK, EDGE_DIM)
    dst_r = dst.reshape(NW, NCHUNK, SCHUNK)
    mesh = plsc.VectorSubcoreMesh(core_axis_name="c", subcore_axis_name="s")
    fn = pl.kernel(
        _scatter_add_body,
        mesh=mesh,
        out_type=jax.ShapeDtypeStruct((NC, N_PAD, 128), jnp.float32),
        scratch_types=[
            pltpu.VMEM((NCHUNK, SCHUNK), jnp.int32),
            pltpu.VMEM((SCHUNK, EDGE_DIM), jnp.float32),
            pltpu.VMEM((SCHUNK, EDGE_DIM), jnp.float32),
            pltpu.VMEM((SCHUNK, 128), jnp.float32),
            pltpu.SemaphoreType.DMA,
            pltpu.SemaphoreType.DMA,
            pltpu.VMEM_SHARED((N_PAD, 128), jnp.float32),
        ],
    )
    return fn(enew_r, dst_r)


def _node_mlp_body(x_ref, a0_ref, a1_ref, w1x_ref, w1a_ref, b1_ref,
                   w2_ref, b2_ref, out_ref):
    agg = a0_ref[...][:, :EDGE_DIM] + a1_ref[...][:, :EDGE_DIM]
    h = (jnp.dot(x_ref[...], w1x_ref[...], preferred_element_type=jnp.float32)
         + jnp.dot(agg, w1a_ref[...], preferred_element_type=jnp.float32)
         + b1_ref[...])
    h = jnp.maximum(h, 0.0)
    out_ref[...] = jnp.dot(h, w2_ref[...],
                           preferred_element_type=jnp.float32) + b2_ref[...]


def _node_mlp(x, a0, a1, w1x, w1a, b1, w2, b2):
    blk = 2000
    grid = N_NODES // blk
    return pl.pallas_call(
        _node_mlp_body,
        grid=(grid,),
        in_specs=[
            pl.BlockSpec((blk, NODE_DIM), lambda i: (i, 0)),
            pl.BlockSpec((blk, 128), lambda i: (i, 0)),
            pl.BlockSpec((blk, 128), lambda i: (i, 0)),
            pl.BlockSpec((NODE_DIM, HIDDEN), lambda i: (0, 0)),
            pl.BlockSpec((EDGE_DIM, HIDDEN), lambda i: (0, 0)),
            pl.BlockSpec((1, HIDDEN), lambda i: (0, 0)),
            pl.BlockSpec((HIDDEN, NODE_DIM), lambda i: (0, 0)),
            pl.BlockSpec((1, NODE_DIM), lambda i: (0, 0)),
        ],
        out_specs=pl.BlockSpec((blk, NODE_DIM), lambda i: (i, 0)),
        out_shape=jax.ShapeDtypeStruct((N_NODES, NODE_DIM), jnp.float32),
    )(x, a0, a1, w1x, w1a, b1, w2, b2)


def kernel(x, edge_index, e, fR_W1, fR_b1, fR_W2, fR_b2,
           fO_W1, fO_b1, fO_W2, fO_b2):
    src = edge_index[0].astype(jnp.int32)
    dst = edge_index[1].astype(jnp.int32)

    w_cat = jnp.concatenate(
        [fR_W1[:NODE_DIM], fR_W1[NODE_DIM:2 * NODE_DIM]], axis=1)
    xd, xs = _precompute(x, w_cat, fR_b1.reshape(1, -1))

    g = _gather_add(xd, xs, dst, src)

    e_new = _edge_mlp(g, e, fR_W1[2 * NODE_DIM:], fR_W2, fR_b2.reshape(1, -1))

    partials = _scatter_add(e_new, dst)

    x_new = _node_mlp(x, partials[0], partials[1],
                      fO_W1[:NODE_DIM], fO_W1[NODE_DIM:],
                      fO_b1.reshape(1, -1), fO_W2, fO_b2.reshape(1, -1))
    return (x_new, e_new)
